# Initial kernel scaffold; baseline (speedup 1.0000x reference)
import jax
import jax.numpy as jnp
from jax.experimental import pallas as pl


def kernel(edge_index, x, W1, b1, W2, b2):
    raise NotImplementedError("write your pallas kernel here")


# speedup vs baseline: 20.8257x; 20.8257x over previous
import functools

import jax
import jax.numpy as jnp
from jax import lax
from jax.experimental import pallas as pl
from jax.experimental.pallas import tpu as pltpu
from jax.experimental.pallas import tpu_sc as plsc

N = 10000
D = 128
E = 320000
NC = 2
NS = 16
NW = NC * NS
C = 125
CPT = (E // NW) // C
STRIPE = 640
LAST = N - 15 * STRIPE

_mesh = plsc.VectorSubcoreMesh(core_axis_name="c", subcore_axis_name="s")


@functools.partial(
    pl.kernel,
    mesh=_mesh,
    out_type=jax.ShapeDtypeStruct((NC * N,), jnp.float32),
    scratch_types=[
        pltpu.VMEM((CPT, C), jnp.int32),
        pltpu.VMEM((C,), jnp.float32),
        pltpu.VMEM((STRIPE,), jnp.float32),
        pltpu.VMEM_SHARED((N,), jnp.float32),
    ],
)
def _sc_deg(col_hbm, zvec_hbm, ones_hbm, out_hbm, cidx, ones_v, stage, dacc):
    c = lax.axis_index("c")
    s = lax.axis_index("s")
    wid = s * NC + c

    pltpu.sync_copy(zvec_hbm, stage)

    @pl.when(s < NS - 1)
    def _():
        base = pl.multiple_of(s * STRIPE, 8)
        pltpu.sync_copy(stage, dacc.at[pl.ds(base, STRIPE)])

    @pl.when(s == NS - 1)
    def _():
        pltpu.sync_copy(stage.at[pl.ds(0, LAST)], dacc.at[pl.ds(15 * STRIPE, LAST)])

    pltpu.sync_copy(ones_hbm, ones_v)
    pltpu.sync_copy(col_hbm.at[pl.ds(wid * CPT, CPT)], cidx)
    plsc.subcore_barrier()

    @pl.loop(0, CPT)
    def _(j):
        pltpu.sync_copy(ones_v, dacc.at[cidx.at[j]], add=True)

    plsc.subcore_barrier()

    @pl.when(s < NS - 1)
    def _():
        base = pl.multiple_of(s * STRIPE, 8)
        obase = pl.multiple_of(c * N + s * STRIPE, 8)
        pltpu.sync_copy(dacc.at[pl.ds(base, STRIPE)], stage)
        pltpu.sync_copy(stage, out_hbm.at[pl.ds(obase, STRIPE)])

    @pl.when(s == NS - 1)
    def _():
        obase = pl.multiple_of(c * N + 15 * STRIPE, 8)
        pltpu.sync_copy(dacc.at[pl.ds(15 * STRIPE, LAST)], stage.at[pl.ds(0, LAST)])
        pltpu.sync_copy(stage.at[pl.ds(0, LAST)], out_hbm.at[pl.ds(obase, LAST)])


@functools.partial(
    pl.kernel,
    mesh=_mesh,
    out_type=jax.ShapeDtypeStruct((NC, N, D), jnp.float32),
    scratch_types=[
        pltpu.VMEM((CPT, C), jnp.int32),
        pltpu.VMEM((CPT, C), jnp.int32),
        pltpu.VMEM((C, D), jnp.float32),
        pltpu.VMEM((C, D), jnp.float32),
        pltpu.VMEM_SHARED((N, D), jnp.float32),
        pltpu.SemaphoreType.DMA,
        pltpu.SemaphoreType.DMA,
    ],
)
def _sc_spmm(row_hbm, col_hbm, h_hbm, zrows_hbm, out_hbm,
             ridx, cidx, bufa, bufb, acc, sema, semb):
    c = lax.axis_index("c")
    s = lax.axis_index("s")
    wid = s * NC + c

    @pl.when(s < NS - 1)
    def _():
        base = pl.multiple_of(s * STRIPE, 8)
        pltpu.sync_copy(zrows_hbm, acc.at[pl.ds(base, STRIPE)])

    @pl.when(s == NS - 1)
    def _():
        pltpu.sync_copy(zrows_hbm.at[pl.ds(0, LAST)], acc.at[pl.ds(15 * STRIPE, LAST)])

    pltpu.sync_copy(row_hbm.at[pl.ds(wid * CPT, CPT)], ridx)
    pltpu.sync_copy(col_hbm.at[pl.ds(wid * CPT, CPT)], cidx)
    plsc.subcore_barrier()

    @pl.loop(0, CPT)
    def _(j):
        pltpu.async_copy(h_hbm.at[ridx.at[j]], bufa, sema).wait()
        pltpu.sync_copy(bufa, acc.at[cidx.at[j]], add=True)

    plsc.subcore_barrier()

    @pl.when(s < NS - 1)
    def _():
        base = pl.multiple_of(s * STRIPE, 8)
        pltpu.sync_copy(acc.at[pl.ds(base, STRIPE)], out_hbm.at[c, pl.ds(base, STRIPE)])

    @pl.when(s == NS - 1)
    def _():
        pltpu.sync_copy(acc.at[pl.ds(15 * STRIPE, LAST)],
                        out_hbm.at[c, pl.ds(15 * STRIPE, LAST)])


BN = 400


def _mm1_body(deg_ref, x_ref, w_ref, o_ref):
    dinv = lax.rsqrt(deg_ref[0] + deg_ref[1] + 1.0)
    o_ref[...] = dinv * jnp.dot(x_ref[...], w_ref[...],
                                preferred_element_type=jnp.float32)


def _mid_body(deg_ref, acc_ref, h_ref, b_ref, w_ref, o_ref):
    dinv = lax.rsqrt(deg_ref[0] + deg_ref[1] + 1.0)
    z = dinv * (acc_ref[0] + acc_ref[1] + h_ref[...]) + b_ref[...]
    z = jnp.maximum(z, 0.0)
    o_ref[...] = dinv * jnp.dot(z, w_ref[...], preferred_element_type=jnp.float32)


def _final_body(deg_ref, acc_ref, h_ref, b_ref, o_ref):
    dinv = lax.rsqrt(deg_ref[0] + deg_ref[1] + 1.0)
    z = dinv * (acc_ref[0] + acc_ref[1] + h_ref[...]) + b_ref[...]
    o_ref[...] = jnp.maximum(z, 0.0)


_deg_spec = pl.BlockSpec((NC, BN, 1), lambda i: (0, i, 0))
_row_spec = pl.BlockSpec((BN, D), lambda i: (i, 0))
_acc_spec = pl.BlockSpec((NC, BN, D), lambda i: (0, i, 0))
_w_spec = pl.BlockSpec((D, D), lambda i: (0, 0))
_b_spec = pl.BlockSpec((1, D), lambda i: (0, 0))

_tc_mm1 = pl.pallas_call(
    _mm1_body,
    grid=(N // BN,),
    in_specs=[_deg_spec, _row_spec, _w_spec],
    out_specs=_row_spec,
    out_shape=jax.ShapeDtypeStruct((N, D), jnp.float32),
)

_tc_mid = pl.pallas_call(
    _mid_body,
    grid=(N // BN,),
    in_specs=[_deg_spec, _acc_spec, _row_spec, _b_spec, _w_spec],
    out_specs=_row_spec,
    out_shape=jax.ShapeDtypeStruct((N, D), jnp.float32),
)

_tc_final = pl.pallas_call(
    _final_body,
    grid=(N // BN,),
    in_specs=[_deg_spec, _acc_spec, _row_spec, _b_spec],
    out_specs=_row_spec,
    out_shape=jax.ShapeDtypeStruct((N, D), jnp.float32),
)


@jax.jit
def kernel(edge_index, x, W1, b1, W2, b2):
    row2d = edge_index[0].astype(jnp.int32).reshape(E // C, C)
    col2d = edge_index[1].astype(jnp.int32).reshape(E // C, C)
    zvec = jnp.zeros((STRIPE,), jnp.float32)
    zrows = jnp.zeros((STRIPE, D), jnp.float32)
    ones = jnp.ones((C,), jnp.float32)

    degp = _sc_deg(col2d, zvec, ones)
    degp3 = degp.reshape(NC, N, 1)

    h1p = _tc_mm1(degp3, x, W1)
    acc1 = _sc_spmm(row2d, col2d, h1p, zrows)
    h2p = _tc_mid(degp3, acc1, h1p, b1.reshape(1, D), W2)
    acc2 = _sc_spmm(row2d, col2d, h2p, zrows)
    z2 = _tc_final(degp3, acc2, h2p, b2.reshape(1, D))
    return z2


# speedup vs baseline: 20.9445x; 1.0057x over previous
import functools

import jax
import jax.numpy as jnp
from jax import lax
from jax.experimental import pallas as pl
from jax.experimental.pallas import tpu as pltpu
from jax.experimental.pallas import tpu_sc as plsc

N = 10000
D = 128
E = 320000
NC = 2
NS = 16
NW = NC * NS
C = 125
CPT = (E // NW) // C
STRIPE = 640
LAST = N - 15 * STRIPE

_mesh = plsc.VectorSubcoreMesh(core_axis_name="c", subcore_axis_name="s")


@functools.partial(
    pl.kernel,
    mesh=_mesh,
    out_type=jax.ShapeDtypeStruct((NC * N,), jnp.float32),
    scratch_types=[
        pltpu.VMEM((CPT, C), jnp.int32),
        pltpu.VMEM((C,), jnp.float32),
        pltpu.VMEM((STRIPE,), jnp.float32),
        pltpu.VMEM_SHARED((N,), jnp.float32),
    ],
)
def _sc_deg(col_hbm, zvec_hbm, ones_hbm, out_hbm, cidx, ones_v, stage, dacc):
    c = lax.axis_index("c")
    s = lax.axis_index("s")
    wid = s * NC + c

    pltpu.sync_copy(zvec_hbm, stage)

    @pl.when(s < NS - 1)
    def _():
        base = pl.multiple_of(s * STRIPE, 8)
        pltpu.sync_copy(stage, dacc.at[pl.ds(base, STRIPE)])

    @pl.when(s == NS - 1)
    def _():
        pltpu.sync_copy(stage.at[pl.ds(0, LAST)], dacc.at[pl.ds(15 * STRIPE, LAST)])

    pltpu.sync_copy(ones_hbm, ones_v)
    pltpu.sync_copy(col_hbm.at[pl.ds(wid * CPT, CPT)], cidx)
    plsc.subcore_barrier()

    @pl.loop(0, CPT)
    def _(j):
        pltpu.sync_copy(ones_v, dacc.at[cidx.at[j]], add=True)

    plsc.subcore_barrier()

    @pl.when(s < NS - 1)
    def _():
        base = pl.multiple_of(s * STRIPE, 8)
        obase = pl.multiple_of(c * N + s * STRIPE, 8)
        pltpu.sync_copy(dacc.at[pl.ds(base, STRIPE)], stage)
        pltpu.sync_copy(stage, out_hbm.at[pl.ds(obase, STRIPE)])

    @pl.when(s == NS - 1)
    def _():
        obase = pl.multiple_of(c * N + 15 * STRIPE, 8)
        pltpu.sync_copy(dacc.at[pl.ds(15 * STRIPE, LAST)], stage.at[pl.ds(0, LAST)])
        pltpu.sync_copy(stage.at[pl.ds(0, LAST)], out_hbm.at[pl.ds(obase, LAST)])


@functools.partial(
    pl.kernel,
    mesh=_mesh,
    out_type=jax.ShapeDtypeStruct((NC, N, D), jnp.float32),
    scratch_types=[
        pltpu.VMEM((CPT, C), jnp.int32),
        pltpu.VMEM((CPT, C), jnp.int32),
        pltpu.VMEM((C, D), jnp.float32),
        pltpu.VMEM((C, D), jnp.float32),
        pltpu.VMEM_SHARED((N, D), jnp.float32),
        pltpu.SemaphoreType.DMA,
        pltpu.SemaphoreType.DMA,
    ],
)
def _sc_spmm(row_hbm, col_hbm, h_hbm, zrows_hbm, dummy_hbm, out_hbm,
             ridx, cidx, bufa, bufb, acc, sga, sgb):
    c = lax.axis_index("c")
    s = lax.axis_index("s")
    wid = s * NC + c

    @pl.when(s < NS - 1)
    def _():
        base = pl.multiple_of(s * STRIPE, 8)
        pltpu.sync_copy(zrows_hbm, acc.at[pl.ds(base, STRIPE)])

    @pl.when(s == NS - 1)
    def _():
        pltpu.sync_copy(zrows_hbm.at[pl.ds(0, LAST)], acc.at[pl.ds(15 * STRIPE, LAST)])

    pltpu.sync_copy(row_hbm.at[pl.ds(wid * CPT, CPT)], ridx)
    pltpu.sync_copy(col_hbm.at[pl.ds(wid * CPT, CPT)], cidx)
    plsc.subcore_barrier()

    @pl.loop(0, CPT)
    def _(j):
        pltpu.async_copy(h_hbm.at[ridx.at[j]], bufa, sga).wait()
        pltpu.sync_copy(bufa, acc.at[cidx.at[j]], add=True)

    plsc.subcore_barrier()

    @pl.when(s < NS - 1)
    def _():
        base = pl.multiple_of(s * STRIPE, 8)
        pltpu.sync_copy(acc.at[pl.ds(base, STRIPE)], out_hbm.at[c, pl.ds(base, STRIPE)])

    @pl.when(s == NS - 1)
    def _():
        pltpu.sync_copy(acc.at[pl.ds(15 * STRIPE, LAST)],
                        out_hbm.at[c, pl.ds(15 * STRIPE, LAST)])


BN = 400


def _mm1_body(deg_ref, x_ref, w_ref, o_ref):
    dinv = lax.rsqrt(deg_ref[0] + deg_ref[1] + 1.0)
    o_ref[...] = dinv * jnp.dot(x_ref[...], w_ref[...],
                                preferred_element_type=jnp.float32)


def _mid_body(deg_ref, acc_ref, h_ref, b_ref, w_ref, o_ref):
    dinv = lax.rsqrt(deg_ref[0] + deg_ref[1] + 1.0)
    z = dinv * (acc_ref[0] + acc_ref[1] + h_ref[...]) + b_ref[...]
    z = jnp.maximum(z, 0.0)
    o_ref[...] = dinv * jnp.dot(z, w_ref[...], preferred_element_type=jnp.float32)


def _final_body(deg_ref, acc_ref, h_ref, b_ref, o_ref):
    dinv = lax.rsqrt(deg_ref[0] + deg_ref[1] + 1.0)
    z = dinv * (acc_ref[0] + acc_ref[1] + h_ref[...]) + b_ref[...]
    o_ref[...] = jnp.maximum(z, 0.0)


_deg_spec = pl.BlockSpec((NC, BN, 1), lambda i: (0, i, 0))
_row_spec = pl.BlockSpec((BN, D), lambda i: (i, 0))
_acc_spec = pl.BlockSpec((NC, BN, D), lambda i: (0, i, 0))
_w_spec = pl.BlockSpec((D, D), lambda i: (0, 0))
_b_spec = pl.BlockSpec((1, D), lambda i: (0, 0))

_tc_mm1 = pl.pallas_call(
    _mm1_body,
    grid=(N // BN,),
    in_specs=[_deg_spec, _row_spec, _w_spec],
    out_specs=_row_spec,
    out_shape=jax.ShapeDtypeStruct((N, D), jnp.float32),
)

_tc_mid = pl.pallas_call(
    _mid_body,
    grid=(N // BN,),
    in_specs=[_deg_spec, _acc_spec, _row_spec, _b_spec, _w_spec],
    out_specs=_row_spec,
    out_shape=jax.ShapeDtypeStruct((N, D), jnp.float32),
)

_tc_final = pl.pallas_call(
    _final_body,
    grid=(N // BN,),
    in_specs=[_deg_spec, _acc_spec, _row_spec, _b_spec],
    out_specs=_row_spec,
    out_shape=jax.ShapeDtypeStruct((N, D), jnp.float32),
)


@jax.jit
def kernel(edge_index, x, W1, b1, W2, b2):
    row2d = edge_index[0].astype(jnp.int32).reshape(E // C, C)
    col2d = edge_index[1].astype(jnp.int32).reshape(E // C, C)
    zvec = jnp.zeros((STRIPE,), jnp.float32)
    zdrain = jnp.zeros((C, D), jnp.float32)
    zrows = jnp.zeros((STRIPE, D), jnp.float32)
    ones = jnp.ones((C,), jnp.float32)

    degp = _sc_deg(col2d, zvec, ones)
    degp3 = degp.reshape(NC, N, 1)

    h1p = _tc_mm1(degp3, x, W1)
    acc1 = _sc_spmm(row2d, col2d, h1p, zrows, zdrain)
    h2p = _tc_mid(degp3, acc1, h1p, b1.reshape(1, D), W2)
    acc2 = _sc_spmm(row2d, col2d, h2p, zrows, zdrain)
    z2 = _tc_final(degp3, acc2, h2p, b2.reshape(1, D))
    return z2
